# Initial kernel scaffold; baseline (speedup 1.0000x reference)
import jax
import jax.numpy as jnp
from jax.experimental import pallas as pl


def kernel(int_edges, nodes, embed, msg_W1, msg_b1, msg_W2, msg_b2, gru_int_k, gru_int_rk, gru_int_b, gru_temp_k, gru_temp_rk, gru_temp_b, ro_W1, ro_b1, ro_W2, ro_b2, ro_W3, ro_b3):
    raise NotImplementedError("write your pallas kernel here")


# speedup vs baseline: 1.3921x; 1.3921x over previous
import functools

import jax
import jax.numpy as jnp
from jax.experimental import pallas as pl
from jax.experimental.pallas import tpu as pltpu

N_NODES = 200
WINDOW = 200
TOTAL = N_NODES * WINDOW
D = 128
T = 2
E = 100000
H1 = 1024


def _msg_mlp(ha, hb, W1a, W1b, b1, W2, b2, *, block):
    n = ha.shape[0]
    assert n % block == 0
    grid = n // block

    def kern(ha_ref, hb_ref, w1a_ref, w1b_ref, b1_ref, w2_ref, b2_ref, o_ref):
        acc = jnp.dot(ha_ref[...], w1a_ref[...], preferred_element_type=jnp.float32)
        acc = acc + jnp.dot(hb_ref[...], w1b_ref[...], preferred_element_type=jnp.float32)
        hid = jnp.maximum(acc + b1_ref[...], 0.0)
        out = jnp.dot(hid, w2_ref[...], preferred_element_type=jnp.float32) + b2_ref[...]
        o_ref[...] = jnp.maximum(out, 0.0)

    return pl.pallas_call(
        kern,
        grid=(grid,),
        in_specs=[
            pl.BlockSpec((block, D), lambda i: (i, 0)),
            pl.BlockSpec((block, D), lambda i: (i, 0)),
            pl.BlockSpec((D, H1), lambda i: (0, 0)),
            pl.BlockSpec((D, H1), lambda i: (0, 0)),
            pl.BlockSpec((1, H1), lambda i: (0, 0)),
            pl.BlockSpec((H1, D), lambda i: (0, 0)),
            pl.BlockSpec((1, D), lambda i: (0, 0)),
        ],
        out_specs=pl.BlockSpec((block, D), lambda i: (i, 0)),
        out_shape=jax.ShapeDtypeStruct((n, D), jnp.float32),
    )(ha, hb, W1a, W1b, b1, W2, b2)


def _gru(sums, counts, h, k, rk, b, *, block):
    n = h.shape[0]
    assert n % block == 0
    grid = n // block

    def kern(s_ref, c_ref, h_ref, k_ref, rk_ref, b_ref, o_ref):
        c = c_ref[...]
        x = jnp.where(c > 0.0, s_ref[...] / jnp.maximum(c, 1.0), 0.0)
        hv = h_ref[...]
        mx = jnp.dot(x, k_ref[...], preferred_element_type=jnp.float32) + b_ref[0:1, :]
        mh = jnp.dot(hv, rk_ref[...], preferred_element_type=jnp.float32) + b_ref[1:2, :]
        z = jax.nn.sigmoid(mx[:, :D] + mh[:, :D])
        r = jax.nn.sigmoid(mx[:, D:2 * D] + mh[:, D:2 * D])
        cand = jnp.tanh(mx[:, 2 * D:] + r * mh[:, 2 * D:])
        o_ref[...] = z * hv + (1.0 - z) * cand

    return pl.pallas_call(
        kern,
        grid=(grid,),
        in_specs=[
            pl.BlockSpec((block, D), lambda i: (i, 0)),
            pl.BlockSpec((block, 1), lambda i: (i, 0)),
            pl.BlockSpec((block, D), lambda i: (i, 0)),
            pl.BlockSpec((D, 3 * D), lambda i: (0, 0)),
            pl.BlockSpec((D, 3 * D), lambda i: (0, 0)),
            pl.BlockSpec((2, 3 * D), lambda i: (0, 0)),
        ],
        out_specs=pl.BlockSpec((block, D), lambda i: (i, 0)),
        out_shape=jax.ShapeDtypeStruct((n, D), jnp.float32),
    )(sums, counts, h, k, rk, b)


def _readout(x, W1, b1, W2, b2, W3p, b3p):
    def kern(x_ref, w1_ref, b1_ref, w2_ref, b2_ref, w3_ref, b3_ref, o_ref):
        a = jnp.maximum(
            jnp.dot(x_ref[...], w1_ref[...], preferred_element_type=jnp.float32)
            + b1_ref[...], 0.0)
        a = jnp.maximum(
            jnp.dot(a, w2_ref[...], preferred_element_type=jnp.float32)
            + b2_ref[...], 0.0)
        lg = jnp.dot(a, w3_ref[...], preferred_element_type=jnp.float32) + b3_ref[...]
        mx = jnp.max(lg, axis=-1, keepdims=True)
        ex = jnp.exp(lg - mx)
        o_ref[...] = ex / jnp.sum(ex, axis=-1, keepdims=True)

    return pl.pallas_call(
        kern,
        out_shape=jax.ShapeDtypeStruct((N_NODES, D), jnp.float32),
    )(x, W1, b1, W2, b2, W3p, b3p)


def kernel(int_edges, nodes, embed, msg_W1, msg_b1, msg_W2, msg_b2,
           gru_int_k, gru_int_rk, gru_int_b, gru_temp_k, gru_temp_rk,
           gru_temp_b, ro_W1, ro_b1, ro_W2, ro_b2, ro_W3, ro_b3):
    node_idx = int_edges[:, 0] * N_NODES + int_edges[:, 1]
    nbr_idx = int_edges[:, 0] * N_NODES + int_edges[:, 2]

    h = jnp.repeat(embed[nodes], WINDOW, axis=0)

    W1a = msg_W1[:D]
    W1b = msg_W1[D:]
    b1r = msg_b1.reshape(1, H1)
    b2r = msg_b2.reshape(1, D)

    io = jnp.arange(TOTAL, dtype=jnp.int32)
    tcount = jnp.where((io >= N_NODES) & (io < TOTAL - N_NODES), 2.0, 1.0)[:, None]

    for _ in range(T):
        ha = jnp.take(h, node_idx, axis=0)
        hb = jnp.take(h, nbr_idx, axis=0)
        m = _msg_mlp(ha, hb, W1a, W1b, b1r, msg_W2, b2r, block=2000)
        sums = jnp.zeros((TOTAL, D), jnp.float32).at[nbr_idx].add(m)
        counts = jnp.zeros((TOTAL, 1), jnp.float32).at[nbr_idx].add(1.0)
        h = _gru(sums, counts, h, gru_int_k, gru_int_rk, gru_int_b, block=2000)

        hs = jnp.roll(h, -N_NODES, axis=0)
        m1 = _msg_mlp(h, hs, W1a, W1b, b1r, msg_W2, b2r, block=2000)
        m2 = _msg_mlp(hs, h, W1a, W1b, b1r, msg_W2, b2r, block=2000)
        z200 = jnp.zeros((N_NODES, D), jnp.float32)
        tsum = (jnp.concatenate([z200, m1[:TOTAL - N_NODES]], axis=0)
                + jnp.concatenate([m2[:TOTAL - N_NODES], z200], axis=0))
        h = _gru(tsum, tcount, h, gru_temp_k, gru_temp_rk, gru_temp_b, block=2000)

    W3p = jnp.zeros((512, D), jnp.float32).at[:, :10].set(ro_W3)
    b3p = jnp.full((1, D), -1e30, jnp.float32).at[0, :10].set(ro_b3)
    probs = _readout(h[:N_NODES], ro_W1, ro_b1.reshape(1, H1),
                     ro_W2, ro_b2.reshape(1, 512), W3p, b3p)
    return probs[:, :10]
